# Initial kernel scaffold; baseline (speedup 1.0000x reference)
import jax
import jax.numpy as jnp
from jax.experimental import pallas as pl


def kernel(input, weights):
    raise NotImplementedError("write your pallas kernel here")


# speedup vs baseline: 1.7204x; 1.7204x over previous
import functools

import jax
import jax.numpy as jnp
from jax import lax
from jax.experimental import pallas as pl
from jax.experimental.pallas import tpu as pltpu
from jax.experimental.pallas import tpu_sc as plsc

PADDING_IDX = 0
LANES = 16
CHUNK = 16


def _make_sc_gather(n_rows, d, seq_len):
    info = plsc.get_sparse_core_info()
    nc, ns = info.num_cores, info.num_subcores
    nw = nc * ns
    assert n_rows % nw == 0
    rows_per_w = n_rows // nw
    assert rows_per_w % CHUNK == 0
    assert seq_len % rows_per_w == 0 or rows_per_w % seq_len == 0
    n_chunks = rows_per_w // CHUNK

    mesh = plsc.VectorSubcoreMesh(core_axis_name="c", subcore_axis_name="s")

    @functools.partial(
        pl.kernel,
        mesh=mesh,
        out_type=jax.ShapeDtypeStruct((n_rows, d), jnp.float32),
        scratch_types=[
            pltpu.VMEM((rows_per_w,), jnp.int32),
            pltpu.VMEM((CHUNK, d), jnp.float32),
            pltpu.SemaphoreType.DMA,
        ],
    )
    def sc_gather(inp_hbm, w_hbm, out_hbm, inp_v, rows_v, sem):
        wid = lax.axis_index("s") * nc + lax.axis_index("c")
        base = wid * rows_per_w
        seq0 = lax.rem(base, seq_len)
        pltpu.sync_copy(inp_hbm.at[pl.ds(base, rows_per_w)], inp_v)
        lane = lax.iota(jnp.int32, LANES)

        def chunk_body(i, carry):
            tok = inp_v[pl.ds(i * CHUNK, CHUNK)]
            pos = jnp.where(
                tok != PADDING_IDX,
                seq0 + i * CHUNK + lane + 1,
                PADDING_IDX,
            )
            pltpu.async_copy(w_hbm.at[pos], rows_v, sem).wait()
            pltpu.sync_copy(rows_v, out_hbm.at[pl.ds(base + i * CHUNK, CHUNK)])
            return carry

        lax.fori_loop(0, n_chunks, chunk_body, 0)

    return sc_gather


def kernel(input, weights):
    bsz, seq_len = input.shape
    d = weights.shape[1]
    n_rows = bsz * seq_len
    sc_gather = _make_sc_gather(n_rows, d, seq_len)
    out = sc_gather(input.reshape(-1), weights)
    return out.reshape(bsz, seq_len, d)
